# baseline (speedup 1.0000x reference)
import functools

import jax
import jax.numpy as jnp
from jax import lax
from jax.experimental import pallas as pl
from jax.experimental.pallas import tpu as pltpu
from jax.experimental.pallas import tpu_sc as plsc

_NC = 2
_NS = 16
_NW = _NC * _NS
_LW = 16


@functools.lru_cache(maxsize=None)
def _make_sc_segment(b, d, t, c):
    n = b * t
    bph = b // _NC
    nph = n // _NC
    rpt = n // _NW
    assert d == _NS * _LW and n % _NW == 0 and b % _NC == 0 and t % _LW == 0
    mesh = plsc.VectorSubcoreMesh(core_axis_name="c", subcore_axis_name="s")

    @functools.partial(
        pl.kernel,
        mesh=mesh,
        compiler_params=pltpu.CompilerParams(
            use_tc_tiling_on_sc=False, needs_layout_passes=False),
        out_type=(
            jax.ShapeDtypeStruct((_NC, d, c), jnp.float32),
            jax.ShapeDtypeStruct((_NW, c), jnp.float32),
            jax.ShapeDtypeStruct((_NW, _LW), jnp.float32),
        ),
        scratch_types=[
            pltpu.VMEM((nph,), jnp.int32),
            pltpu.VMEM((bph, _LW, t + 1), jnp.float32),
            pltpu.VMEM((_LW, c + 1), jnp.float32),
            pltpu.VMEM((_LW, c + 1), jnp.float32),
            pltpu.VMEM((_LW,), jnp.float32),
            pltpu.SemaphoreType.DMA,
        ] + [pltpu.SemaphoreType.DMA] * bph,
    )
    def sc_segment(feat_hbm, lab_hbm, s_out, cnt_out, sq_out,
                   idx_v, slab_v, acc_v, cnt_v, sq_v, sem_lab, *sem_slab):
        ci = lax.axis_index("c")
        si = lax.axis_index("s")
        wid = si * _NC + ci
        cp_lab = pltpu.async_copy(
            lab_hbm.at[pl.ds(ci * nph, nph)], idx_v, sem_lab)
        cp_slab = [
            pltpu.async_copy(
                feat_hbm.at[pl.ds(ci * bph + bb, 1), pl.ds(si * _LW, _LW), :],
                slab_v.at[pl.ds(bb, 1), :, pl.ds(0, t)], sem_slab[bb])
            for bb in range(bph)
        ]

        d_iota = lax.iota(jnp.int32, _LW)
        ones16 = jnp.full((_LW,), 1.0, jnp.float32)
        zeros16f = jnp.zeros((_LW,), jnp.float32)

        def z_body(k, _):
            for dd in range(_LW):
                acc_v[dd, pl.ds(k * _LW, _LW)] = zeros16f
            cnt_v[0, pl.ds(k * _LW, _LW)] = zeros16f
            return 0

        lax.fori_loop(0, c // _LW, z_body, 0)
        cp_lab.wait()

        sq = (zeros16f, zeros16f, zeros16f, zeros16f)
        for b_loc in range(bph):
            cp_slab[b_loc].wait()
            b_idx = jnp.full((_LW,), b_loc, jnp.int32)

            def t_body(tt, sq, b_idx=b_idx, b_loc=b_loc):
                labs = idx_v[pl.ds(b_loc * t + tt * _LW, _LW)]
                t_base = jnp.full((_LW,), tt * _LW, jnp.int32)
                vals = [plsc.load_gather(slab_v, [b_idx, d_iota, t_base + j])
                        for j in range(_LW)]
                cls = [jnp.full((_LW,), labs[j], jnp.int32)
                       for j in range(_LW)]
                for j in range(_LW):
                    plsc.addupdate_scatter(acc_v, [d_iota, cls[j]], vals[j])
                sq = list(sq)
                for j in range(_LW):
                    sq[j % 4] = sq[j % 4] + vals[j] * vals[j]
                return tuple(sq)

            sq = lax.fori_loop(0, t // _LW, t_body, sq)
        sq_v[...] = sq[0] + sq[1] + sq[2] + sq[3]

        def cnt_body(tt, _):
            labs = idx_v[pl.ds(si * rpt + tt * _LW, _LW)]
            cls = [jnp.full((_LW,), labs[j], jnp.int32) for j in range(_LW)]
            for j in range(_LW):
                plsc.addupdate_scatter(cnt_v, [d_iota, cls[j]], ones16)
            return 0

        lax.fori_loop(0, rpt // _LW, cnt_body, 0)

        pltpu.sync_copy(acc_v.at[:, pl.ds(0, c)],
                        s_out.at[ci, pl.ds(si * _LW, _LW), :])
        pltpu.sync_copy(cnt_v.at[0, pl.ds(0, c)], cnt_out.at[wid])
        pltpu.sync_copy(sq_v, sq_out.at[wid])

    return sc_segment


@functools.lru_cache(maxsize=None)
def _make_finish(b, d, t, c):
    n = b * t

    def body(sp_ref, cnt_ref, cen_ref, sq_ref, loss_ref, diff_ref):
        s = jnp.transpose(sp_ref[0] + sp_ref[1])
        counts = jnp.sum(cnt_ref[...], axis=1, keepdims=True)
        cen = cen_ref[...]
        diff_ref[...] = (counts * cen - s) / jnp.maximum(counts, 1.0)
        sumf2 = jnp.sum(sq_ref[...])
        cross = jnp.sum(s * cen)
        sumc2 = jnp.sum(counts * (cen * cen))
        loss = (sumf2 - 2.0 * cross + sumc2) / (n * d)
        loss_ref[...] = jnp.broadcast_to(loss, (1, 1))

    return pl.pallas_call(
        body,
        out_shape=(
            jax.ShapeDtypeStruct((1, 1), jnp.float32),
            jax.ShapeDtypeStruct((c, d), jnp.float32),
        ),
    )


def kernel(feature, centers, label):
    b, d, t = feature.shape
    c = centers.shape[0]
    lab = label.reshape(-1)
    s_part, cnt_part, sq_part = _make_sc_segment(b, d, t, c)(feature, lab)
    loss, difference = _make_finish(b, d, t, c)(
        s_part, jnp.transpose(cnt_part), centers, sq_part)
    return loss[0, 0], difference


# speedup vs baseline: 1.0039x; 1.0039x over previous
import functools

import jax
import jax.numpy as jnp
from jax import lax
from jax.experimental import pallas as pl
from jax.experimental.pallas import tpu as pltpu
from jax.experimental.pallas import tpu_sc as plsc

_NC = 2
_NS = 16
_NW = _NC * _NS
_LW = 16


@functools.lru_cache(maxsize=None)
def _make_sc_segment(b, d, t, c):
    n = b * t
    bph = b // _NC
    nph = n // _NC
    rpt = n // _NW
    assert d == _NS * _LW and n % _NW == 0 and b % _NC == 0 and t % _LW == 0
    mesh = plsc.VectorSubcoreMesh(core_axis_name="c", subcore_axis_name="s")

    @functools.partial(
        pl.kernel,
        mesh=mesh,
        compiler_params=pltpu.CompilerParams(
            use_tc_tiling_on_sc=False, needs_layout_passes=False),
        out_type=(
            jax.ShapeDtypeStruct((_NC, d, c), jnp.float32),
            jax.ShapeDtypeStruct((_NW, c), jnp.float32),
            jax.ShapeDtypeStruct((_NW, _LW), jnp.float32),
        ),
        scratch_types=[
            pltpu.VMEM((nph,), jnp.int32),
            pltpu.VMEM((bph, _LW, t + 1), jnp.float32),
            pltpu.VMEM((_LW, c + 1), jnp.float32),
            pltpu.VMEM((_LW, c + 1), jnp.float32),
            pltpu.VMEM((_LW,), jnp.float32),
            pltpu.SemaphoreType.DMA,
        ] + [pltpu.SemaphoreType.DMA] * bph,
    )
    def sc_segment(feat_hbm, lab_hbm, s_out, cnt_out, sq_out,
                   idx_v, slab_v, acc_v, cnt_v, sq_v, sem_lab, *sem_slab):
        ci = lax.axis_index("c")
        si = lax.axis_index("s")
        wid = si * _NC + ci
        cp_lab = pltpu.async_copy(
            lab_hbm.at[pl.ds(ci * nph, nph)], idx_v, sem_lab)
        cp_slab = [
            pltpu.async_copy(
                feat_hbm.at[pl.ds(ci * bph + bb, 1), pl.ds(si * _LW, _LW), :],
                slab_v.at[pl.ds(bb, 1), :, pl.ds(0, t)], sem_slab[bb])
            for bb in range(bph)
        ]

        d_iota = lax.iota(jnp.int32, _LW)
        ones16 = jnp.full((_LW,), 1.0, jnp.float32)
        zeros16f = jnp.zeros((_LW,), jnp.float32)

        def z_body(k, _):
            for dd in range(_LW):
                acc_v[dd, pl.ds(k * _LW, _LW)] = zeros16f
            cnt_v[0, pl.ds(k * _LW, _LW)] = zeros16f
            return 0

        lax.fori_loop(0, c // _LW, z_body, 0)
        cp_lab.wait()

        def cnt_body(tt, _):
            labs = idx_v[pl.ds(si * rpt + tt * _LW, _LW)]
            cls = [jnp.full((_LW,), labs[j], jnp.int32) for j in range(_LW)]
            for j in range(_LW):
                plsc.addupdate_scatter(cnt_v, [d_iota, cls[j]], ones16)
            return 0

        lax.fori_loop(0, rpt // _LW, cnt_body, 0)

        sq = (zeros16f, zeros16f, zeros16f, zeros16f)
        for b_loc in range(bph):
            cp_slab[b_loc].wait()
            b_idx = jnp.full((_LW,), b_loc, jnp.int32)

            def t_body(tt, sq, b_idx=b_idx, b_loc=b_loc):
                labs = idx_v[pl.ds(b_loc * t + tt * _LW, _LW)]
                t_base = jnp.full((_LW,), tt * _LW, jnp.int32)
                vals = [plsc.load_gather(slab_v, [b_idx, d_iota, t_base + j])
                        for j in range(_LW)]
                cls = [jnp.full((_LW,), labs[j], jnp.int32)
                       for j in range(_LW)]
                for j in range(_LW):
                    plsc.addupdate_scatter(acc_v, [d_iota, cls[j]], vals[j])
                sq = list(sq)
                for j in range(_LW):
                    sq[j % 4] = sq[j % 4] + vals[j] * vals[j]
                return tuple(sq)

            sq = lax.fori_loop(0, t // _LW, t_body, sq)
        sq_v[...] = sq[0] + sq[1] + sq[2] + sq[3]

        pltpu.sync_copy(acc_v.at[:, pl.ds(0, c)],
                        s_out.at[ci, pl.ds(si * _LW, _LW), :])
        pltpu.sync_copy(cnt_v.at[0, pl.ds(0, c)], cnt_out.at[wid])
        pltpu.sync_copy(sq_v, sq_out.at[wid])

    return sc_segment


@functools.lru_cache(maxsize=None)
def _make_finish(b, d, t, c):
    n = b * t

    def body(sp_ref, cnt_ref, cen_ref, sq_ref, loss_ref, diff_ref):
        s = jnp.transpose(sp_ref[0] + sp_ref[1])
        counts = jnp.sum(cnt_ref[...], axis=1, keepdims=True)
        cen = cen_ref[...]
        diff_ref[...] = (counts * cen - s) / jnp.maximum(counts, 1.0)
        sumf2 = jnp.sum(sq_ref[...])
        cross = jnp.sum(s * cen)
        sumc2 = jnp.sum(counts * (cen * cen))
        loss = (sumf2 - 2.0 * cross + sumc2) / (n * d)
        loss_ref[...] = jnp.broadcast_to(loss, (1, 1))

    return pl.pallas_call(
        body,
        out_shape=(
            jax.ShapeDtypeStruct((1, 1), jnp.float32),
            jax.ShapeDtypeStruct((c, d), jnp.float32),
        ),
    )


def kernel(feature, centers, label):
    b, d, t = feature.shape
    c = centers.shape[0]
    lab = label.reshape(-1)
    s_part, cnt_part, sq_part = _make_sc_segment(b, d, t, c)(feature, lab)
    loss, difference = _make_finish(b, d, t, c)(
        s_part, jnp.transpose(cnt_part), centers, sq_part)
    return loss[0, 0], difference
